# baseline (speedup 1.0000x reference)
import functools

import jax
import jax.numpy as jnp
from jax.experimental import pallas as pl
from jax.experimental.pallas import tpu as pltpu

N = 10000
D_IN = 128
D_OUT = 128
BLK = 200


def _decoder_kernel(x_ref, w_ref, b_ref, a_ref, out_ref,
                    support, gcn, m, m_old, z, acc, *, nsteps):
    i = pl.program_id(0)

    @pl.when(i == 0)
    def _init():
        support[...] = jnp.dot(x_ref[...], w_ref[...],
                               preferred_element_type=jnp.float32)
        m[...] = jnp.full_like(m[...], -jnp.inf)

    g = jnp.dot(a_ref[...], support[...],
                preferred_element_type=jnp.float32) + b_ref[...]
    gcn[pl.ds(i * BLK, BLK), :] = g
    m[...] = jnp.maximum(m[...], jnp.max(g, axis=0, keepdims=True))

    @pl.when(i == nsteps - 3)
    def _partial_flush():
        m_old[...] = m[...]
        e = jnp.exp(gcn[: N - 2 * BLK, :] - m[...])
        z[...] = jnp.sum(e, axis=0, keepdims=True)
        acc[...] = jax.lax.dot_general(
            e, x_ref[: N - 2 * BLK, :], (((0,), (0,)), ((), ())),
            preferred_element_type=jnp.float32)

    @pl.when(i >= nsteps - 2)
    def _incremental_flush():
        m_new = m[...]
        alpha = jnp.exp(m_old[...] - m_new)
        e_blk = jnp.exp(g - m_new)
        upd = jax.lax.dot_general(
            e_blk, x_ref[pl.ds(i * BLK, BLK), :], (((0,), (0,)), ((), ())),
            preferred_element_type=jnp.float32)
        z_new = z[...] * alpha + jnp.sum(e_blk, axis=0, keepdims=True)
        acc_new = acc[...] * alpha.T + upd
        z[...] = z_new
        acc[...] = acc_new
        m_old[...] = m_new

        @pl.when(i == nsteps - 1)
        def _emit():
            out_ref[...] = acc_new / z_new.T


def kernel(main_feat, main_adj, W, b):
    nsteps = N // BLK
    b2d = b.reshape(1, D_OUT)
    out = pl.pallas_call(
        functools.partial(_decoder_kernel, nsteps=nsteps),
        grid=(nsteps,),
        in_specs=[
            pl.BlockSpec((N, D_IN), lambda i: (0, 0)),
            pl.BlockSpec((D_IN, D_OUT), lambda i: (0, 0)),
            pl.BlockSpec((1, D_OUT), lambda i: (0, 0)),
            pl.BlockSpec((BLK, N), lambda i: (i, 0)),
        ],
        out_specs=pl.BlockSpec((D_OUT, D_IN), lambda i: (0, 0)),
        out_shape=jax.ShapeDtypeStruct((D_OUT, D_IN), jnp.float32),
        scratch_shapes=[
            pltpu.VMEM((N, D_OUT), jnp.float32),
            pltpu.VMEM((N, D_OUT), jnp.float32),
            pltpu.VMEM((1, D_OUT), jnp.float32),
            pltpu.VMEM((1, D_OUT), jnp.float32),
            pltpu.VMEM((1, D_OUT), jnp.float32),
            pltpu.VMEM((D_OUT, D_IN), jnp.float32),
        ],
        compiler_params=pltpu.CompilerParams(
            dimension_semantics=("arbitrary",),
        ),
    )(main_feat, W, b2d, main_adj)
    return out
